# baseline (speedup 1.0000x reference)
import jax
import jax.numpy as jnp
from jax import lax
from jax.experimental import pallas as pl
from jax.experimental.pallas import tpu as pltpu
from jax.experimental.pallas import tpu_sc as plsc

_N = 27554
_K = 9
_NC = 2
_NS = 16
_NW = _NC * _NS
_L = 16
_NTAB = 28672
_CHUNK = _NTAB // _NS
_H = _CHUNK // 2
_GCH = _NTAB // _NW
_NV = _GCH // _L
_SCALE = 1.0 / (3.0 * _N)


def _lap_body(big_hbm, out_hbm,
              buf8, bufd, tab, bufg,
              stage_v, part_v, spall, sp_part, sem_in, sem_tab):
    c = lax.axis_index("c")
    s = lax.axis_index("s")
    wid = c * _NS + s
    tbase = s * _CHUNK
    gbase = wid * _GCH

    cp_g = pltpu.async_copy(big_hbm.at[:, pl.ds(gbase, _GCH)], bufg, sem_in)

    for p in range(2):
        pltpu.sync_copy(big_hbm.at[pl.ds(0, 8), pl.ds(tbase + p * _H, _H)],
                        buf8)

        for cc in range(3):
            def _sub(i, cc=cc):
                off = i * _L
                bufd[pl.ds(cc * _H + off, _L)] = (
                    buf8[cc, pl.ds(off, _L)] - buf8[3 + cc, pl.ds(off, _L)])
            plsc.parallel_loop(0, _H // _L, 1, unroll=4)(_sub)

        cps = [pltpu.async_copy(
                   bufd.at[pl.ds(cc * _H, _H)],
                   spall.at[pl.ds(cc * _NTAB + tbase + p * _H, _H)],
                   sem_tab)
               for cc in range(3)]
        for cp in cps:
            cp.wait()
    plsc.subcore_barrier()

    pltpu.sync_copy(spall, tab)
    cp_g.wait()

    def _gather(j, acc):
        off = j * _L
        iks = [plsc.bitcast(bufg[8 + k, pl.ds(off, _L)], jnp.int32)
               for k in range(_K)]
        comp = []
        for cc in range(3):
            base = cc * _NTAB
            g = [plsc.load_gather(tab, [ik + base if cc else ik])
                 for ik in iks]
            t01 = g[0] + g[1]
            t23 = g[2] + g[3]
            t45 = g[4] + g[5]
            t67 = g[6] + g[7]
            comp.append(((t01 + t23) + (t45 + t67)) + g[8])
        rw = 1.0 / jnp.maximum(bufg[6, pl.ds(off, _L)], 1e-30)
        ex = tab[pl.ds(gbase + off, _L)] - comp[0] * rw
        ey = tab[pl.ds(_NTAB + gbase + off, _L)] - comp[1] * rw
        ez = tab[pl.ds(2 * _NTAB + gbase + off, _L)] - comp[2] * rw
        return acc + (ex * ex + ey * ey + ez * ez) * bufg[7, pl.ds(off, _L)]

    acc = plsc.parallel_loop(0, _NV, 1, unroll=2,
                             carry=jnp.zeros((_L,), jnp.float32))(_gather)

    stage_v[...] = acc
    pltpu.sync_copy(stage_v, sp_part.at[pl.ds(s * _L, _L)])
    plsc.subcore_barrier()

    @pl.when(s == 0)
    def _():
        pltpu.sync_copy(sp_part, part_v)
        tot = part_v[pl.ds(0, _L)]
        for t in range(1, _NS):
            tot = tot + part_v[pl.ds(t * _L, _L)]
        total = jnp.sum(tot) * _SCALE
        stage_v[...] = jnp.broadcast_to(total, (_L,))
        pltpu.sync_copy(stage_v.at[pl.ds(0, 8)], out_hbm.at[pl.ds(c * 8, 8)])


_lap_call = pl.kernel(
    _lap_body,
    out_type=jax.ShapeDtypeStruct((_L,), jnp.float32),
    mesh=plsc.VectorSubcoreMesh(core_axis_name="c", subcore_axis_name="s",
                                num_cores=_NC),
    compiler_params=pltpu.CompilerParams(needs_layout_passes=False),
    scratch_types=[
        pltpu.VMEM((8, _H), jnp.float32),
        pltpu.VMEM((3 * _H,), jnp.float32),
        pltpu.VMEM((3 * _NTAB,), jnp.float32),
        pltpu.VMEM((17, _GCH), jnp.float32),
        pltpu.VMEM((_L,), jnp.float32),
        pltpu.VMEM((_NS * _L,), jnp.float32),
        pltpu.VMEM_SHARED((3 * _NTAB,), jnp.float32),
        pltpu.VMEM_SHARED((_NS * _L,), jnp.float32),
        pltpu.SemaphoreType.DMA,
        pltpu.SemaphoreType.DMA,
    ],
)


def kernel(v_1, v_2, adj_indices, adj_weights, laplace_w):
    pad = _NTAB - _N
    idx_f = jax.lax.bitcast_convert_type(
        adj_indices.astype(jnp.int32)[:, :_K].T, jnp.float32)
    big = jnp.pad(
        jnp.concatenate([v_1.astype(jnp.float32).T,
                         v_2.astype(jnp.float32).T,
                         adj_weights.astype(jnp.float32).T,
                         laplace_w.astype(jnp.float32).T,
                         idx_f], axis=0),
        ((0, 0), (0, pad)))
    out = _lap_call(big)
    return out[0] + out[8]


# speedup vs baseline: 1.0526x; 1.0526x over previous
import jax
import jax.numpy as jnp
from jax import lax
from jax.experimental import pallas as pl
from jax.experimental.pallas import tpu as pltpu
from jax.experimental.pallas import tpu_sc as plsc

_N = 27554
_K = 9
_NC = 2
_NS = 16
_NW = _NC * _NS
_L = 16
_NTAB = 28672
_CHUNK = _NTAB // _NS
_H = _CHUNK // 2
_GCH = _NTAB // _NW
_NV = _GCH // _L
_SCALE = 1.0 / (3.0 * _N)


def _lap_body(big_hbm, idx_hbm, out_hbm,
              buf8, bufd, tab, bufg, idx_v,
              stage_v, part_v, spall, sp_part, sem_in, sem_tab, sem_pull):
    c = lax.axis_index("c")
    s = lax.axis_index("s")
    wid = c * _NS + s
    gbase = wid * _GCH

    cp_idx = pltpu.async_copy(idx_hbm.at[:, pl.ds(gbase, _GCH)], idx_v, sem_in)
    cp_g = pltpu.async_copy(big_hbm.at[:, pl.ds(gbase, _GCH)], bufg, sem_in)

    _HALF = _NTAB // 2
    pulls = []
    for p in range(2):
        pbase = p * _HALF + s * _H
        pltpu.sync_copy(big_hbm.at[:, pl.ds(pbase, _H)], buf8)

        for cc in range(3):
            def _sub(i, cc=cc):
                off = i * _L
                bufd[pl.ds(cc * _H + off, _L)] = (
                    buf8[cc, pl.ds(off, _L)] - buf8[3 + cc, pl.ds(off, _L)])
            plsc.parallel_loop(0, _H // _L, 1, unroll=4)(_sub)

        cps = [pltpu.async_copy(
                   bufd.at[pl.ds(cc * _H, _H)],
                   spall.at[pl.ds(cc * _NTAB + pbase, _H)],
                   sem_tab)
               for cc in range(3)]
        for cp in cps:
            cp.wait()
        plsc.subcore_barrier()
        pulls += [pltpu.async_copy(
                      spall.at[pl.ds(cc * _NTAB + p * _HALF, _HALF)],
                      tab.at[pl.ds(cc * _NTAB + p * _HALF, _HALF)],
                      sem_pull)
                  for cc in range(3)]
    for cp in pulls:
        cp.wait()
    cp_idx.wait()
    cp_g.wait()

    def _gather(j, acc):
        off = j * _L
        iks = [idx_v[k, pl.ds(off, _L)] for k in range(_K)]
        comp = []
        for cc in range(3):
            base = cc * _NTAB
            g = [plsc.load_gather(tab, [ik + base if cc else ik])
                 for ik in iks]
            t01 = g[0] + g[1]
            t23 = g[2] + g[3]
            t45 = g[4] + g[5]
            t67 = g[6] + g[7]
            comp.append(((t01 + t23) + (t45 + t67)) + g[8])
        rw = 1.0 / jnp.maximum(bufg[6, pl.ds(off, _L)], 1e-30)
        ex = tab[pl.ds(gbase + off, _L)] - comp[0] * rw
        ey = tab[pl.ds(_NTAB + gbase + off, _L)] - comp[1] * rw
        ez = tab[pl.ds(2 * _NTAB + gbase + off, _L)] - comp[2] * rw
        return acc + (ex * ex + ey * ey + ez * ez) * bufg[7, pl.ds(off, _L)]

    acc = plsc.parallel_loop(0, _NV, 1, unroll=2,
                             carry=jnp.zeros((_L,), jnp.float32))(_gather)

    stage_v[...] = acc
    pltpu.sync_copy(stage_v, sp_part.at[pl.ds(s * _L, _L)])
    plsc.subcore_barrier()

    @pl.when(s == 0)
    def _():
        pltpu.sync_copy(sp_part, part_v)
        tot = part_v[pl.ds(0, _L)]
        for t in range(1, _NS):
            tot = tot + part_v[pl.ds(t * _L, _L)]
        total = jnp.sum(tot) * _SCALE
        stage_v[...] = jnp.broadcast_to(total, (_L,))
        pltpu.sync_copy(stage_v.at[pl.ds(0, 8)], out_hbm.at[pl.ds(c * 8, 8)])


_lap_call = pl.kernel(
    _lap_body,
    out_type=jax.ShapeDtypeStruct((_L,), jnp.float32),
    mesh=plsc.VectorSubcoreMesh(core_axis_name="c", subcore_axis_name="s",
                                num_cores=_NC),
    compiler_params=pltpu.CompilerParams(needs_layout_passes=False),
    scratch_types=[
        pltpu.VMEM((8, _H), jnp.float32),
        pltpu.VMEM((3 * _H,), jnp.float32),
        pltpu.VMEM((3 * _NTAB,), jnp.float32),
        pltpu.VMEM((8, _GCH), jnp.float32),
        pltpu.VMEM((_K, _GCH), jnp.int32),
        pltpu.VMEM((_L,), jnp.float32),
        pltpu.VMEM((_NS * _L,), jnp.float32),
        pltpu.VMEM_SHARED((3 * _NTAB,), jnp.float32),
        pltpu.VMEM_SHARED((_NS * _L,), jnp.float32),
        pltpu.SemaphoreType.DMA,
        pltpu.SemaphoreType.DMA,
        pltpu.SemaphoreType.DMA,
    ],
)


def kernel(v_1, v_2, adj_indices, adj_weights, laplace_w):
    pad = _NTAB - _N
    big = jnp.pad(
        jnp.concatenate([v_1.astype(jnp.float32).T,
                         v_2.astype(jnp.float32).T,
                         adj_weights.astype(jnp.float32).T,
                         laplace_w.astype(jnp.float32).T], axis=0),
        ((0, 0), (0, pad)))
    idxp = jnp.pad(adj_indices.astype(jnp.int32)[:, :_K].T, ((0, 0), (0, pad)))
    out = _lap_call(big, idxp)
    return out[0] + out[8]
